# baseline (speedup 1.0000x reference)
import math

import jax
import jax.numpy as jnp
from jax.experimental import pallas as pl
from jax.experimental.pallas import tpu as pltpu

BQ = 512
HP = 4


def _attn_block_kernel(kv_ref, q_ref, m_ref, o_ref, vb_ref, kn_ref):
    hpdh = q_ref.shape[-1]
    dh = hpdh // HP
    s_len = kv_ref.shape[1]

    @pl.when(pl.program_id(1) == 0)
    def _init():
        v = kv_ref[1]
        ecol = (jax.lax.broadcasted_iota(jnp.int32, (s_len, dh), 1) == 0
                ).astype(jnp.bfloat16)
        k = kv_ref[0].astype(jnp.float32)
        for j in range(HP):
            vb_ref[:, 2 * j * dh:(2 * j + 1) * dh] = v[:, j * dh:(j + 1) * dh]
            vb_ref[:, (2 * j + 1) * dh:(2 * j + 2) * dh] = ecol
            kj = k[:, j * dh:(j + 1) * dh]
            kn_ref[j] = jnp.sqrt(jnp.max(jnp.sum(kj * kj, axis=-1)))

    qp = q_ref[0]
    q32 = qp.astype(jnp.float32)
    outs = []
    for j in range(HP):
        qj32 = q32[:, j * dh:(j + 1) * dh]
        qn = jnp.sqrt(jnp.sum(qj32 * qj32, axis=-1, keepdims=True))
        bound = qn * kn_ref[j]
        qj = qp[:, j * dh:(j + 1) * dh]
        kj = kv_ref[0, :, j * dh:(j + 1) * dh]
        s2 = jax.lax.dot_general(
            qj, kj, (((1,), (1,)), ((), ())), preferred_element_type=jnp.float32
        )
        p = jnp.exp2(s2 - bound) * m_ref[j]
        oe = jax.lax.dot_general(
            p.astype(jnp.bfloat16), vb_ref[:, 2 * j * dh:(2 * j + 2) * dh],
            (((1,), (0,)), ((), ())), preferred_element_type=jnp.float32,
        )
        outs.append(oe[:, 0:dh] / oe[:, dh:dh + 1])
    o_ref[:, 0, :] = jnp.concatenate(outs, axis=-1)


def kernel(query, key, value, mask):
    b, s, h, dh = query.shape
    hd = h * dh
    nq = s // BQ
    nh = h // HP
    c = math.log2(math.e) / math.sqrt(dh)

    kvq = jnp.stack([
        key.reshape(s, hd),
        value.reshape(s, hd),
        query.reshape(s, hd) * c,
    ]).astype(jnp.bfloat16)

    out = pl.pallas_call(
        _attn_block_kernel,
        grid=(nh, nq),
        in_specs=[
            pl.BlockSpec((2, s, HP * dh), lambda hh, i: (0, 0, hh)),
            pl.BlockSpec((1, BQ, HP * dh), lambda hh, i: (2, i, hh)),
            pl.BlockSpec((HP, BQ, s), lambda hh, i: (hh, i, 0)),
        ],
        out_specs=pl.BlockSpec((BQ, 1, HP * dh), lambda hh, i: (i, 0, hh)),
        out_shape=jax.ShapeDtypeStruct((s, b, hd), jnp.float32),
        scratch_shapes=[
            pltpu.VMEM((s, HP * 2 * dh), jnp.bfloat16),
            pltpu.SMEM((HP,), jnp.float32),
        ],
    )(kvq, kvq, mask)

    return out


# speedup vs baseline: 1.1730x; 1.1730x over previous
import math

import jax
import jax.numpy as jnp
from jax.experimental import pallas as pl
from jax.experimental.pallas import tpu as pltpu

BQ = 512
HP = 4


def _attn_block_kernel(q_ref, k_ref, v_ref, m_ref, o_ref, vb_ref, kn_ref):
    hpdh = q_ref.shape[-1]
    dh = hpdh // HP
    s_len = k_ref.shape[0]

    @pl.when(pl.program_id(1) == 0)
    def _init():
        v = v_ref[...]
        ecol = (jax.lax.broadcasted_iota(jnp.int32, (s_len, dh), 1) == 0
                ).astype(jnp.bfloat16)
        k = k_ref[...].astype(jnp.float32)
        for j in range(HP):
            vb_ref[:, 2 * j * dh:(2 * j + 1) * dh] = v[:, j * dh:(j + 1) * dh]
            vb_ref[:, (2 * j + 1) * dh:(2 * j + 2) * dh] = ecol
            kj = k[:, j * dh:(j + 1) * dh]
            kn_ref[j] = jnp.sqrt(jnp.max(jnp.sum(kj * kj, axis=-1)))

    qp = q_ref[...]
    q32 = qp.astype(jnp.float32)
    outs = []
    for j in range(HP):
        qj32 = q32[:, j * dh:(j + 1) * dh]
        qn = jnp.sqrt(jnp.sum(qj32 * qj32, axis=-1, keepdims=True))
        bound = qn * kn_ref[j]
        qj = qp[:, j * dh:(j + 1) * dh]
        kj = k_ref[:, j * dh:(j + 1) * dh]
        s2 = jax.lax.dot_general(
            qj, kj, (((1,), (1,)), ((), ())), preferred_element_type=jnp.float32
        )
        p = jnp.exp2(s2 - bound) * m_ref[j]
        oe = jax.lax.dot_general(
            p.astype(jnp.bfloat16), vb_ref[:, 2 * j * dh:(2 * j + 2) * dh],
            (((1,), (0,)), ((), ())), preferred_element_type=jnp.float32,
        )
        outs.append(oe[:, 0:dh] / oe[:, dh:dh + 1])
    o_ref[:, 0, :] = jnp.concatenate(outs, axis=-1)


def kernel(query, key, value, mask):
    b, s, h, dh = query.shape
    hd = h * dh
    nq = s // BQ
    nh = h // HP
    c = math.log2(math.e) / math.sqrt(dh)

    qb = (query.reshape(s, hd) * c).astype(jnp.bfloat16)
    kb = key.reshape(s, hd).astype(jnp.bfloat16)
    vb = value.reshape(s, hd).astype(jnp.bfloat16)

    out = pl.pallas_call(
        _attn_block_kernel,
        grid=(nh, nq),
        in_specs=[
            pl.BlockSpec((BQ, HP * dh), lambda hh, i: (i, hh)),
            pl.BlockSpec((s, HP * dh), lambda hh, i: (0, hh)),
            pl.BlockSpec((s, HP * dh), lambda hh, i: (0, hh)),
            pl.BlockSpec((HP, BQ, s), lambda hh, i: (hh, i, 0)),
        ],
        out_specs=pl.BlockSpec((BQ, 1, HP * dh), lambda hh, i: (i, 0, hh)),
        out_shape=jax.ShapeDtypeStruct((s, b, hd), jnp.float32),
        scratch_shapes=[
            pltpu.VMEM((s, HP * 2 * dh), jnp.bfloat16),
            pltpu.SMEM((HP,), jnp.float32),
        ],
    )(qb, kb, vb, mask)

    return out


# speedup vs baseline: 1.1764x; 1.0029x over previous
import math

import jax
import jax.numpy as jnp
from jax.experimental import pallas as pl
from jax.experimental.pallas import tpu as pltpu

BQ = 1024
HP = 2


def _attn_block_kernel(q_ref, k_ref, v_ref, m_ref, o_ref, vb_ref, kn_ref):
    hpdh = q_ref.shape[-1]
    dh = hpdh // HP
    s_len = k_ref.shape[0]

    @pl.when(pl.program_id(1) == 0)
    def _init():
        v = v_ref[...]
        ecol = (jax.lax.broadcasted_iota(jnp.int32, (s_len, dh), 1) == 0
                ).astype(jnp.bfloat16)
        k = k_ref[...].astype(jnp.float32)
        for j in range(HP):
            vb_ref[:, 2 * j * dh:(2 * j + 1) * dh] = v[:, j * dh:(j + 1) * dh]
            vb_ref[:, (2 * j + 1) * dh:(2 * j + 2) * dh] = ecol
            kj = k[:, j * dh:(j + 1) * dh]
            kn_ref[j] = jnp.sqrt(jnp.max(jnp.sum(kj * kj, axis=-1)))

    qp = q_ref[...]
    q32 = qp.astype(jnp.float32)
    outs = []
    for j in range(HP):
        qj32 = q32[:, j * dh:(j + 1) * dh]
        qn = jnp.sqrt(jnp.sum(qj32 * qj32, axis=-1, keepdims=True))
        bound = qn * kn_ref[j]
        qj = qp[:, j * dh:(j + 1) * dh]
        kj = k_ref[:, j * dh:(j + 1) * dh]
        s2 = jax.lax.dot_general(
            qj, kj, (((1,), (1,)), ((), ())), preferred_element_type=jnp.float32
        )
        p = jnp.exp2(s2 - bound) * m_ref[j]
        oe = jax.lax.dot_general(
            p.astype(jnp.bfloat16), vb_ref[:, 2 * j * dh:(2 * j + 2) * dh],
            (((1,), (0,)), ((), ())), preferred_element_type=jnp.float32,
        )
        outs.append(oe[:, 0:dh] / oe[:, dh:dh + 1])
    o_ref[:, 0, :] = jnp.concatenate(outs, axis=-1)


def kernel(query, key, value, mask):
    b, s, h, dh = query.shape
    hd = h * dh
    nq = s // BQ
    nh = h // HP
    c = math.log2(math.e) / math.sqrt(dh)

    qb = (query.reshape(s, hd) * c).astype(jnp.bfloat16)
    kb = key.reshape(s, hd).astype(jnp.bfloat16)
    vb = value.reshape(s, hd).astype(jnp.bfloat16)

    out = pl.pallas_call(
        _attn_block_kernel,
        grid=(nh, nq),
        in_specs=[
            pl.BlockSpec((BQ, HP * dh), lambda hh, i: (i, hh)),
            pl.BlockSpec((s, HP * dh), lambda hh, i: (0, hh)),
            pl.BlockSpec((s, HP * dh), lambda hh, i: (0, hh)),
            pl.BlockSpec((HP, BQ, s), lambda hh, i: (hh, i, 0)),
        ],
        out_specs=pl.BlockSpec((BQ, 1, HP * dh), lambda hh, i: (i, 0, hh)),
        out_shape=jax.ShapeDtypeStruct((s, b, hd), jnp.float32),
        scratch_shapes=[
            pltpu.VMEM((s, HP * 2 * dh), jnp.bfloat16),
            pltpu.SMEM((HP,), jnp.float32),
        ],
    )(qb, kb, vb, mask)

    return out
